# baseline (speedup 1.0000x reference)
import functools

import jax
import jax.numpy as jnp
from jax import lax
from jax.experimental import pallas as pl
from jax.experimental.pallas import tpu as pltpu
from jax.experimental.pallas import tpu_sc as plsc

_N = 100000
_H = 128
_ZP = 128
_G = 1024

_NW = 32
_KC = 28
_L = 112
_NP = _NW * _KC * _L

_R = 16
_B = _R * 128
_GRID = _NP // _B


def _tc_body(z_ref, px_ref, py_ref, pz_ref, embt_ref, wpt_ref, b1_ref,
             wout_ref, y_ref):
    i = pl.program_id(0)
    embt = embt_ref[...]
    wp0 = wpt_ref[:, 0:1]
    wp1 = wpt_ref[:, 1:2]
    wp2 = wpt_ref[:, 2:3]
    b1 = b1_ref[...]
    wout = wout_ref[...]
    rowi = lax.broadcasted_iota(jnp.int32, (_ZP, 128), 0)
    lane = lax.broadcasted_iota(jnp.int32, (1, 128), 1)

    for g in range(_R):
        zg = z_ref[g:g + 1, :]
        oh = (rowi == zg).astype(jnp.float32)
        xt = jnp.dot(embt, oh, preferred_element_type=jnp.float32,
                     precision=lax.Precision.HIGHEST)
        xt = xt + wp0 * px_ref[g:g + 1, :]
        xt = xt + wp1 * py_ref[g:g + 1, :]
        xt = xt + wp2 * pz_ref[g:g + 1, :]
        xt = xt + b1
        xt = xt * jax.nn.sigmoid(xt)
        yg = jnp.sum(xt * wout, axis=0, keepdims=True)
        atom = (i * _R + g) * 128 + lane
        y_ref[g:g + 1, :] = jnp.where(atom < _N, yg, 0.0)


def _sc_body(y_ref, b_ref, out_ref, bidx_v, y_v, zbuf, acc_sh, sem):
    c = lax.axis_index("c")
    s = lax.axis_index("s")
    w = c * 16 + s

    pltpu.sync_copy(b_ref.at[w], bidx_v)
    pltpu.sync_copy(y_ref.at[w], y_v)

    @pl.when(s == 0)
    def _():
        for j in range(_G // 16):
            zbuf[pl.ds(j * 16, 16)] = jnp.zeros((16,), jnp.float32)
        pltpu.sync_copy(zbuf, acc_sh)

    plsc.subcore_barrier()

    for k in range(_KC):
        pltpu.sync_copy(y_v.at[k], acc_sh.at[bidx_v.at[k]], add=True)

    plsc.subcore_barrier()

    @pl.when(s == 0)
    def _():
        pltpu.sync_copy(acc_sh, out_ref.at[c])


@jax.jit
def kernel(z, pos, batch, embed, Wpos, W1, b1, Wout):
    pad = _NP - _N
    zi = jnp.concatenate([z.astype(jnp.int32), jnp.zeros((pad,), jnp.int32)])
    z2 = zi.reshape(_NP // 128, 128)
    posp = jnp.concatenate([pos, jnp.zeros((pad, 3), jnp.float32)], axis=0)
    px2 = posp[:, 0].reshape(_NP // 128, 128)
    py2 = posp[:, 1].reshape(_NP // 128, 128)
    pz2 = posp[:, 2].reshape(_NP // 128, 128)
    embp = jnp.zeros((_ZP, _H), jnp.float32).at[:embed.shape[0]].set(embed)
    emb2t = jnp.dot(embp, W1, precision=lax.Precision.HIGHEST).T
    wp2t = jnp.dot(Wpos, W1, precision=lax.Precision.HIGHEST).T

    y2 = pl.pallas_call(
        _tc_body,
        grid=(_GRID,),
        in_specs=[
            pl.BlockSpec((_R, 128), lambda i: (i, 0)),
            pl.BlockSpec((_R, 128), lambda i: (i, 0)),
            pl.BlockSpec((_R, 128), lambda i: (i, 0)),
            pl.BlockSpec((_R, 128), lambda i: (i, 0)),
            pl.BlockSpec((_H, _ZP), lambda i: (0, 0)),
            pl.BlockSpec((_H, 3), lambda i: (0, 0)),
            pl.BlockSpec((_H, 1), lambda i: (0, 0)),
            pl.BlockSpec((_H, 1), lambda i: (0, 0)),
        ],
        out_specs=pl.BlockSpec((_R, 128), lambda i: (i, 0)),
        out_shape=jax.ShapeDtypeStruct((_NP // 128, 128), jnp.float32),
        compiler_params=pltpu.CompilerParams(
            dimension_semantics=("parallel",)),
    )(z2, px2, py2, pz2, emb2t, wp2t, b1.reshape(_H, 1), Wout)

    batch_p = jnp.concatenate(
        [batch.astype(jnp.int32), jnp.zeros((pad,), jnp.int32)])
    y3 = y2.reshape(_NW, _KC, _L)
    b3 = batch_p.reshape(_NW, _KC, _L)

    seg_sum = pl.kernel(
        _sc_body,
        out_type=jax.ShapeDtypeStruct((2, _G), jnp.float32),
        mesh=plsc.VectorSubcoreMesh(core_axis_name="c", subcore_axis_name="s"),
        scratch_types=[
            pltpu.VMEM((_KC, _L), jnp.int32),
            pltpu.VMEM((_KC, _L), jnp.float32),
            pltpu.VMEM((_G,), jnp.float32),
            pltpu.VMEM_SHARED((_G,), jnp.float32),
            pltpu.SemaphoreType.DMA,
        ],
    )(y3, b3)

    return (seg_sum[0] + seg_sum[1]).reshape(_G, 1)


# speedup vs baseline: 1.2287x; 1.2287x over previous
import functools

import jax
import jax.numpy as jnp
from jax import lax
from jax.experimental import pallas as pl
from jax.experimental.pallas import tpu as pltpu
from jax.experimental.pallas import tpu_sc as plsc

_N = 100000
_H = 128
_ZP = 128
_G = 1024

_NW = 32
_KC = 28
_L = 112
_NP = _NW * _KC * _L

_R = 16
_B = _R * 128
_GRID = _NP // _B


def _tc_body(z_ref, px_ref, py_ref, pz_ref, hi_ref, lo_ref, wpt_ref, b1_ref,
             wout_ref, y_ref):
    i = pl.program_id(0)
    wp0 = wpt_ref[:, 0:1]
    wp1 = wpt_ref[:, 1:2]
    wp2 = wpt_ref[:, 2:3]
    b1 = b1_ref[...]
    wout = wout_ref[...]
    rowi = lax.broadcasted_iota(jnp.int32, (_ZP, _B), 0)
    lane = lax.broadcasted_iota(jnp.int32, (1, _B), 1)

    zg = z_ref[...].reshape(1, _B)
    oh = (rowi == zg).astype(jnp.bfloat16)
    xt = jnp.dot(hi_ref[...], oh, preferred_element_type=jnp.float32)
    xt = xt + jnp.dot(lo_ref[...], oh, preferred_element_type=jnp.float32)
    xt = xt + wp0 * px_ref[...].reshape(1, _B)
    xt = xt + wp1 * py_ref[...].reshape(1, _B)
    xt = xt + wp2 * pz_ref[...].reshape(1, _B)
    xt = xt + b1
    xt = xt * jax.nn.sigmoid(xt)
    yg = jnp.sum(xt * wout, axis=0, keepdims=True)
    atom = i * _B + lane
    y_ref[...] = jnp.where(atom < _N, yg, 0.0).reshape(_R, 128)


def _sc_body(y_ref, b_ref, out_ref, bidx_v, y_v, zbuf, acc_sh):
    c = lax.axis_index("c")
    s = lax.axis_index("s")
    w = c * 16 + s

    pltpu.sync_copy(b_ref.at[w], bidx_v)
    pltpu.sync_copy(y_ref.at[w], y_v)

    @pl.when(s == 0)
    def _():
        for j in range(_G // 16):
            zbuf[pl.ds(j * 16, 16)] = jnp.zeros((16,), jnp.float32)
        pltpu.sync_copy(zbuf, acc_sh)

    plsc.subcore_barrier()

    for k in range(_KC):
        pltpu.sync_copy(y_v.at[k], acc_sh.at[bidx_v.at[k]], add=True)

    plsc.subcore_barrier()

    @pl.when(s == 0)
    def _():
        pltpu.sync_copy(acc_sh, out_ref.at[c])


@jax.jit
def kernel(z, pos, batch, embed, Wpos, W1, b1, Wout):
    pad = _NP - _N
    zi = jnp.concatenate([z.astype(jnp.int32), jnp.zeros((pad,), jnp.int32)])
    z2 = zi.reshape(_NP // 128, 128)
    posp = jnp.concatenate([pos, jnp.zeros((pad, 3), jnp.float32)], axis=0)
    px2 = posp[:, 0].reshape(_NP // 128, 128)
    py2 = posp[:, 1].reshape(_NP // 128, 128)
    pz2 = posp[:, 2].reshape(_NP // 128, 128)
    embp = jnp.zeros((_ZP, _H), jnp.float32).at[:embed.shape[0]].set(embed)
    emb2t = jnp.dot(embp, W1, precision=lax.Precision.HIGHEST).T
    wp2t = jnp.dot(Wpos, W1, precision=lax.Precision.HIGHEST).T
    emb2t_hi = emb2t.astype(jnp.bfloat16)
    emb2t_lo = (emb2t - emb2t_hi.astype(jnp.float32)).astype(jnp.bfloat16)

    y2 = pl.pallas_call(
        _tc_body,
        grid=(_GRID,),
        in_specs=[
            pl.BlockSpec((_R, 128), lambda i: (i, 0)),
            pl.BlockSpec((_R, 128), lambda i: (i, 0)),
            pl.BlockSpec((_R, 128), lambda i: (i, 0)),
            pl.BlockSpec((_R, 128), lambda i: (i, 0)),
            pl.BlockSpec((_H, _ZP), lambda i: (0, 0)),
            pl.BlockSpec((_H, _ZP), lambda i: (0, 0)),
            pl.BlockSpec((_H, 3), lambda i: (0, 0)),
            pl.BlockSpec((_H, 1), lambda i: (0, 0)),
            pl.BlockSpec((_H, 1), lambda i: (0, 0)),
        ],
        out_specs=pl.BlockSpec((_R, 128), lambda i: (i, 0)),
        out_shape=jax.ShapeDtypeStruct((_NP // 128, 128), jnp.float32),
        compiler_params=pltpu.CompilerParams(
            dimension_semantics=("parallel",)),
    )(z2, px2, py2, pz2, emb2t_hi, emb2t_lo, wp2t, b1.reshape(_H, 1), Wout)

    batch_p = jnp.concatenate(
        [batch.astype(jnp.int32), jnp.zeros((pad,), jnp.int32)])
    y3 = y2.reshape(_NW, _KC, _L)
    b3 = batch_p.reshape(_NW, _KC, _L)

    seg_sum = pl.kernel(
        _sc_body,
        out_type=jax.ShapeDtypeStruct((2, _G), jnp.float32),
        mesh=plsc.VectorSubcoreMesh(core_axis_name="c", subcore_axis_name="s"),
        scratch_types=[
            pltpu.VMEM((_KC, _L), jnp.int32),
            pltpu.VMEM((_KC, _L), jnp.float32),
            pltpu.VMEM((_G,), jnp.float32),
            pltpu.VMEM_SHARED((_G,), jnp.float32),
        ],
    )(y3, b3)

    return (seg_sum[0] + seg_sum[1]).reshape(_G, 1)


# speedup vs baseline: 1.3284x; 1.0812x over previous
import functools

import jax
import jax.numpy as jnp
from jax import lax
from jax.experimental import pallas as pl
from jax.experimental.pallas import tpu as pltpu
from jax.experimental.pallas import tpu_sc as plsc

_N = 100000
_H = 128
_ZP = 128
_G = 1024

_NW = 32
_KC = 28
_L = 112
_NP = _NW * _KC * _L

_R = 56
_B = _R * 128
_GRID = _NP // _B


def _tc_body(z_ref, px_ref, py_ref, pz_ref, hi_ref, lo_ref, wpt_ref, b1_ref,
             wout_ref, y_ref):
    i = pl.program_id(0)
    wp0 = wpt_ref[:, 0:1]
    wp1 = wpt_ref[:, 1:2]
    wp2 = wpt_ref[:, 2:3]
    b1 = b1_ref[...]
    wout = wout_ref[...]
    rowi = lax.broadcasted_iota(jnp.int32, (_ZP, _B), 0)
    lane = lax.broadcasted_iota(jnp.int32, (1, _B), 1)

    zg = z_ref[...].reshape(1, _B)
    oh = (rowi == zg).astype(jnp.bfloat16)
    xt = jnp.dot(hi_ref[...], oh, preferred_element_type=jnp.float32)
    xt = xt + jnp.dot(lo_ref[...], oh, preferred_element_type=jnp.float32)
    xt = xt + wp0 * px_ref[...].reshape(1, _B)
    xt = xt + wp1 * py_ref[...].reshape(1, _B)
    xt = xt + wp2 * pz_ref[...].reshape(1, _B)
    xt = xt + b1
    xt = xt * jax.nn.sigmoid(xt)
    yg = jnp.sum(xt * wout, axis=0, keepdims=True)
    atom = i * _B + lane
    y_ref[...] = jnp.where(atom < _N, yg, 0.0).reshape(_R, 128)


def _sc_body(y_ref, b_ref, out_ref, bidx_v, y_v, zbuf, acc_sh, sem):
    c = lax.axis_index("c")
    s = lax.axis_index("s")
    w = c * 16 + s

    pltpu.sync_copy(b_ref.at[w], bidx_v)
    pltpu.sync_copy(y_ref.at[w], y_v)

    @pl.when(s == 0)
    def _():
        for j in range(_G // 16):
            zbuf[pl.ds(j * 16, 16)] = jnp.zeros((16,), jnp.float32)
        pltpu.sync_copy(zbuf, acc_sh)

    plsc.subcore_barrier()

    copies = [
        pltpu.async_copy(y_v.at[k], acc_sh.at[bidx_v.at[k]], sem, add=True)
        for k in range(_KC)
    ]
    for cp in copies:
        cp.wait()

    plsc.subcore_barrier()

    @pl.when(s == 0)
    def _():
        pltpu.sync_copy(acc_sh, out_ref.at[c])


@jax.jit
def kernel(z, pos, batch, embed, Wpos, W1, b1, Wout):
    pad = _NP - _N
    zi = jnp.concatenate([z.astype(jnp.int32), jnp.zeros((pad,), jnp.int32)])
    z2 = zi.reshape(_NP // 128, 128)
    posp = jnp.concatenate([pos, jnp.zeros((pad, 3), jnp.float32)], axis=0)
    px2 = posp[:, 0].reshape(_NP // 128, 128)
    py2 = posp[:, 1].reshape(_NP // 128, 128)
    pz2 = posp[:, 2].reshape(_NP // 128, 128)
    embp = jnp.zeros((_ZP, _H), jnp.float32).at[:embed.shape[0]].set(embed)
    emb2t = jnp.dot(embp, W1, precision=lax.Precision.HIGHEST).T
    wp2t = jnp.dot(Wpos, W1, precision=lax.Precision.HIGHEST).T
    emb2t_hi = emb2t.astype(jnp.bfloat16)
    emb2t_lo = (emb2t - emb2t_hi.astype(jnp.float32)).astype(jnp.bfloat16)

    y2 = pl.pallas_call(
        _tc_body,
        grid=(_GRID,),
        in_specs=[
            pl.BlockSpec((_R, 128), lambda i: (i, 0)),
            pl.BlockSpec((_R, 128), lambda i: (i, 0)),
            pl.BlockSpec((_R, 128), lambda i: (i, 0)),
            pl.BlockSpec((_R, 128), lambda i: (i, 0)),
            pl.BlockSpec((_H, _ZP), lambda i: (0, 0)),
            pl.BlockSpec((_H, _ZP), lambda i: (0, 0)),
            pl.BlockSpec((_H, 3), lambda i: (0, 0)),
            pl.BlockSpec((_H, 1), lambda i: (0, 0)),
            pl.BlockSpec((_H, 1), lambda i: (0, 0)),
        ],
        out_specs=pl.BlockSpec((_R, 128), lambda i: (i, 0)),
        out_shape=jax.ShapeDtypeStruct((_NP // 128, 128), jnp.float32),
        compiler_params=pltpu.CompilerParams(
            dimension_semantics=("parallel",)),
    )(z2, px2, py2, pz2, emb2t_hi, emb2t_lo, wp2t, b1.reshape(_H, 1), Wout)

    batch_p = jnp.concatenate(
        [batch.astype(jnp.int32), jnp.zeros((pad,), jnp.int32)])
    y3 = y2.reshape(_NW, _KC, _L)
    b3 = batch_p.reshape(_NW, _KC, _L)

    seg_sum = pl.kernel(
        _sc_body,
        out_type=jax.ShapeDtypeStruct((2, _G), jnp.float32),
        mesh=plsc.VectorSubcoreMesh(core_axis_name="c", subcore_axis_name="s"),
        scratch_types=[
            pltpu.VMEM((_KC, _L), jnp.int32),
            pltpu.VMEM((_KC, _L), jnp.float32),
            pltpu.VMEM((_G,), jnp.float32),
            pltpu.VMEM_SHARED((_G,), jnp.float32),
            pltpu.SemaphoreType.DMA,
        ],
    )(y3, b3)

    return (seg_sum[0] + seg_sum[1]).reshape(_G, 1)


# speedup vs baseline: 1.3391x; 1.0081x over previous
import functools

import jax
import jax.numpy as jnp
from jax import lax
from jax.experimental import pallas as pl
from jax.experimental.pallas import tpu as pltpu
from jax.experimental.pallas import tpu_sc as plsc

_N = 100000
_H = 128
_ZP = 128
_G = 1024

_NW = 32
_KC = 28
_L = 112
_NP = _NW * _KC * _L

_R = 112
_B = _R * 128
_GRID = _NP // _B


def _tc_body(z_ref, px_ref, py_ref, pz_ref, hi_ref, lo_ref, wpt_ref, b1_ref,
             wout_ref, y_ref):
    i = pl.program_id(0)
    wp0 = wpt_ref[:, 0:1]
    wp1 = wpt_ref[:, 1:2]
    wp2 = wpt_ref[:, 2:3]
    b1 = b1_ref[...]
    wout = wout_ref[...]
    rowi = lax.broadcasted_iota(jnp.int32, (_ZP, _B), 0)
    lane = lax.broadcasted_iota(jnp.int32, (1, _B), 1)

    zg = z_ref[...].reshape(1, _B)
    oh = (rowi == zg).astype(jnp.bfloat16)
    xt = jnp.dot(hi_ref[...], oh, preferred_element_type=jnp.float32)
    xt = xt + jnp.dot(lo_ref[...], oh, preferred_element_type=jnp.float32)
    xt = xt + wp0 * px_ref[...].reshape(1, _B)
    xt = xt + wp1 * py_ref[...].reshape(1, _B)
    xt = xt + wp2 * pz_ref[...].reshape(1, _B)
    xt = xt + b1
    xt = xt * jax.nn.sigmoid(xt)
    yg = jnp.sum(xt * wout, axis=0, keepdims=True)
    atom = i * _B + lane
    y_ref[...] = jnp.where(atom < _N, yg, 0.0).reshape(_R, 128)


def _sc_body(y_ref, b_ref, out_ref, bidx_v, y_v, zbuf, acc_sh, sem):
    c = lax.axis_index("c")
    s = lax.axis_index("s")
    w = c * 16 + s

    pltpu.sync_copy(b_ref.at[w], bidx_v)
    pltpu.sync_copy(y_ref.at[w], y_v)

    @pl.when(s == 0)
    def _():
        for j in range(_G // 16):
            zbuf[pl.ds(j * 16, 16)] = jnp.zeros((16,), jnp.float32)
        pltpu.sync_copy(zbuf, acc_sh)

    plsc.subcore_barrier()

    copies = [
        pltpu.async_copy(y_v.at[k], acc_sh.at[bidx_v.at[k]], sem, add=True)
        for k in range(_KC)
    ]
    for cp in copies:
        cp.wait()

    plsc.subcore_barrier()

    @pl.when(s == 0)
    def _():
        pltpu.sync_copy(acc_sh, out_ref.at[c])


@jax.jit
def kernel(z, pos, batch, embed, Wpos, W1, b1, Wout):
    pad = _NP - _N
    zi = jnp.concatenate([z.astype(jnp.int32), jnp.zeros((pad,), jnp.int32)])
    z2 = zi.reshape(_NP // 128, 128)
    posp = jnp.concatenate([pos, jnp.zeros((pad, 3), jnp.float32)], axis=0)
    px2 = posp[:, 0].reshape(_NP // 128, 128)
    py2 = posp[:, 1].reshape(_NP // 128, 128)
    pz2 = posp[:, 2].reshape(_NP // 128, 128)
    embp = jnp.zeros((_ZP, _H), jnp.float32).at[:embed.shape[0]].set(embed)
    emb2t = jnp.dot(embp, W1, precision=lax.Precision.HIGHEST).T
    wp2t = jnp.dot(Wpos, W1, precision=lax.Precision.HIGHEST).T
    emb2t_hi = emb2t.astype(jnp.bfloat16)
    emb2t_lo = (emb2t - emb2t_hi.astype(jnp.float32)).astype(jnp.bfloat16)

    y2 = pl.pallas_call(
        _tc_body,
        grid=(_GRID,),
        in_specs=[
            pl.BlockSpec((_R, 128), lambda i: (i, 0)),
            pl.BlockSpec((_R, 128), lambda i: (i, 0)),
            pl.BlockSpec((_R, 128), lambda i: (i, 0)),
            pl.BlockSpec((_R, 128), lambda i: (i, 0)),
            pl.BlockSpec((_H, _ZP), lambda i: (0, 0)),
            pl.BlockSpec((_H, _ZP), lambda i: (0, 0)),
            pl.BlockSpec((_H, 3), lambda i: (0, 0)),
            pl.BlockSpec((_H, 1), lambda i: (0, 0)),
            pl.BlockSpec((_H, 1), lambda i: (0, 0)),
        ],
        out_specs=pl.BlockSpec((_R, 128), lambda i: (i, 0)),
        out_shape=jax.ShapeDtypeStruct((_NP // 128, 128), jnp.float32),
        compiler_params=pltpu.CompilerParams(
            dimension_semantics=("parallel",)),
    )(z2, px2, py2, pz2, emb2t_hi, emb2t_lo, wp2t, b1.reshape(_H, 1), Wout)

    batch_p = jnp.concatenate(
        [batch.astype(jnp.int32), jnp.zeros((pad,), jnp.int32)])
    y3 = y2.reshape(_NW, _KC, _L)
    b3 = batch_p.reshape(_NW, _KC, _L)

    seg_sum = pl.kernel(
        _sc_body,
        out_type=jax.ShapeDtypeStruct((2, _G), jnp.float32),
        mesh=plsc.VectorSubcoreMesh(core_axis_name="c", subcore_axis_name="s"),
        scratch_types=[
            pltpu.VMEM((_KC, _L), jnp.int32),
            pltpu.VMEM((_KC, _L), jnp.float32),
            pltpu.VMEM((_G,), jnp.float32),
            pltpu.VMEM_SHARED((_G,), jnp.float32),
            pltpu.SemaphoreType.DMA,
        ],
    )(y3, b3)

    return (seg_sum[0] + seg_sum[1]).reshape(_G, 1)
